# Initial kernel scaffold; baseline (speedup 1.0000x reference)
import jax
import jax.numpy as jnp
from jax.experimental import pallas as pl


def kernel(x, edge_index, edge_attr, batch, r_target, W1l, b1l, W1r, b1r, W1e, a1, c1, W2l, b2l, W2r, b2r, W2e, a2, c2, W3l, b3l, W3r, b3r, W3e, a3, c3, Wt, bt):
    raise NotImplementedError("write your pallas kernel here")


# speedup vs baseline: 1.0921x; 1.0921x over previous
import functools

import jax
import jax.numpy as jnp
from jax.experimental import pallas as pl
from jax.experimental.pallas import tpu as pltpu

N = 10000
E = 320000
G = 64
C = 128
N_PAD = 10240
ROW_BLK = 512


def _mm_body(x_ref, w_ref, b_ref, o_ref):
    o_ref[...] = (
        jnp.dot(x_ref[...], w_ref[...], preferred_element_type=jnp.float32)
        + b_ref[...]
    )


def _project(h, W, b):
    K = W.shape[1]
    grid = (N_PAD // ROW_BLK,)
    return pl.pallas_call(
        _mm_body,
        grid=grid,
        in_specs=[
            pl.BlockSpec((ROW_BLK, 128), lambda i: (i, 0)),
            pl.BlockSpec((128, K), lambda i: (0, 0)),
            pl.BlockSpec((1, K), lambda i: (0, 0)),
        ],
        out_specs=pl.BlockSpec((ROW_BLK, K), lambda i: (i, 0)),
        out_shape=jax.ShapeDtypeStruct((N_PAD, K), jnp.float32),
    )(h, W, b.reshape(1, K))


def _gatv2_layer(h, src, dst, ea, Wl, bl, Wr, br, We, att, bias):
    W = jnp.concatenate([Wl, Wr], axis=1)
    b = jnp.concatenate([bl, br], axis=0)
    proj = _project(h, W, b)
    xl = proj[:N, :128]
    xr = proj[:N, 128:]
    ef = ea @ We
    e = jax.nn.leaky_relu(xl[src] + xr[dst] + ef, negative_slope=0.2)
    logits = jnp.sum(e * att.reshape(1, 128), axis=-1)
    m = jax.ops.segment_max(logits, dst, num_segments=N)
    m = jnp.where(jnp.isfinite(m), m, 0.0)
    ex = jnp.exp(logits - m[dst])
    denom = jax.ops.segment_sum(ex, dst, num_segments=N)
    alpha = ex / (denom[dst] + 1e-16)
    out = jax.ops.segment_sum(alpha[:, None] * xl[src], dst, num_segments=N)
    return out + bias


def kernel(x, edge_index, edge_attr, batch, r_target, W1l, b1l, W1r, b1r, W1e, a1, c1, W2l, b2l, W2r, b2r, W2e, a2, c2, W3l, b3l, W3r, b3r, W3e, a3, c3, Wt, bt):
    dst0 = edge_index[1]
    deg = jax.ops.segment_sum(jnp.ones((E,), jnp.float32), dst0, num_segments=N)
    mean_attr = jax.ops.segment_sum(edge_attr, dst0, num_segments=N) / jnp.clip(deg, 1.0)[:, None]
    loop = jnp.arange(N, dtype=edge_index.dtype)
    src = jnp.concatenate([edge_index[0], loop])
    dst = jnp.concatenate([dst0, loop])
    ea = jnp.concatenate([edge_attr, mean_attr], axis=0)

    def pad(h):
        return jnp.pad(h, ((0, N_PAD - N), (0, 0)))

    h = pad(x)
    o1 = _gatv2_layer(h, src, dst, ea, W1l, b1l, W1r, b1r, W1e, a1.reshape(-1), c1)
    h = pad(jax.nn.relu(o1))
    o2 = _gatv2_layer(h, src, dst, ea, W2l, b2l, W2r, b2r, W2e, a2.reshape(-1), c2)
    h = pad(jax.nn.relu(o2))
    o3 = _gatv2_layer(h, src, dst, ea, W3l, b3l, W3r, b3r, W3e, a3.reshape(-1), c3)

    cnt = jax.ops.segment_sum(jnp.ones((N,), jnp.float32), batch, num_segments=G)
    g = jax.ops.segment_sum(o3, batch, num_segments=G) / jnp.clip(cnt, 1.0)[:, None]
    return g @ Wt + bt


# speedup vs baseline: 2.9307x; 2.6835x over previous
import functools

import jax
import jax.numpy as jnp
from jax import lax
from jax.experimental import pallas as pl
from jax.experimental.pallas import tpu as pltpu
from jax.experimental.pallas import tpu_sc as plsc

N = 10000
E = 320000
G = 64
C = 128
NT = 4

NC = 2
NS = 16
NW = NC * NS
L = 16

N_PAD = 10240
SL = N_PAD // NW
CH = 128

E_AUG = E + N
NCH = -(-E_AUG // (NW * CH))
EPT = NCH * CH
E_PAD = EPT * NW

NCH0 = -(-E // (NW * CH))
E0PT = NCH0 * CH
E0_PAD = E0PT * NW

PADN = N + 16

_MESH = plsc.VectorSubcoreMesh(core_axis_name="c", subcore_axis_name="s",
                               num_cores=NC, num_subcores=NS)
_SC_PARAMS = pltpu.CompilerParams(needs_layout_passes=False)

_i32 = jnp.int32
_f32 = jnp.float32


def _wid():
    return lax.axis_index("s") * NC + lax.axis_index("c")


def _iota16():
    return lax.iota(_i32, L)


def _fill(ref, n_vregs, value, base=0):
    def body(v, _):
        ref[pl.ds(base + v * L, L)] = jnp.full((L,), value, ref.dtype)
        return ()
    lax.fori_loop(0, n_vregs, body, ())


@functools.partial(
    pl.kernel,
    out_type=(
        jax.ShapeDtypeStruct((NW * N_PAD,), _f32),
        jax.ShapeDtypeStruct((NW * 4 * N_PAD,), _f32),
    ),
    mesh=_MESH,
    compiler_params=_SC_PARAMS,
    scratch_types=[
        pltpu.VMEM((N_PAD,), _f32),
        pltpu.VMEM((4 * N_PAD,), _f32),
        pltpu.VMEM((CH,), _i32),
        pltpu.VMEM((CH * 4,), _f32),
    ],
)
def _k0(dst_hbm, ea_hbm, deg_out, asum_out, degl, al, dbuf, eabuf):
    w = _wid()
    iota = _iota16()
    _fill(degl, N_PAD // L, 0.0)
    _fill(al, 4 * N_PAD // L, 0.0)

    def chunk(ch, _):
        base = pl.multiple_of(w * E0PT + ch * CH, CH)
        pltpu.sync_copy(dst_hbm.at[pl.ds(base, CH)], dbuf)
        pltpu.sync_copy(ea_hbm.at[pl.ds(base * 4, CH * 4)], eabuf)

        def group(g, _):
            d16 = dbuf[pl.ds(g * L, L)]
            plsc.addupdate_scatter(degl, [d16], jnp.full((L,), 1.0, _f32))
            e4 = (g * L + iota) * 4
            for k in range(4):
                eak = plsc.load_gather(eabuf, [e4 + k])
                plsc.addupdate_scatter(al, [d16 + k * N_PAD], eak)
            return ()
        lax.fori_loop(0, CH // L, group, ())
        return ()
    lax.fori_loop(0, NCH0, chunk, ())

    pltpu.sync_copy(degl, deg_out.at[pl.ds(w * N_PAD, N_PAD)])
    pltpu.sync_copy(al, asum_out.at[pl.ds(w * 4 * N_PAD, 4 * N_PAD)])


@functools.partial(
    pl.kernel,
    out_type=jax.ShapeDtypeStruct((N_PAD * 4,), _f32),
    mesh=_MESH,
    compiler_params=_SC_PARAMS,
    scratch_types=[
        pltpu.VMEM((SL,), _f32),
        pltpu.VMEM((4 * SL,), _f32),
        pltpu.VMEM((SL,), _f32),
        pltpu.VMEM((4 * SL,), _f32),
        pltpu.VMEM((SL * 4,), _f32),
    ],
)
def _k0b(deg_part, asum_part, mean_out, accd, acca, dbuf, abuf, obuf):
    w = _wid()
    iota = _iota16()
    _fill(accd, SL // L, 0.0)
    _fill(acca, 4 * SL // L, 0.0)

    def part(j, _):
        pltpu.sync_copy(deg_part.at[pl.ds(j * N_PAD + w * SL, SL)], dbuf)
        for k in range(4):
            pltpu.sync_copy(
                asum_part.at[pl.ds(j * 4 * N_PAD + k * N_PAD + w * SL, SL)],
                abuf.at[pl.ds(k * SL, SL)])

        def acc(v, _):
            s = pl.ds(v * L, L)
            accd[s] = accd[s] + dbuf[s]
            return ()
        lax.fori_loop(0, SL // L, acc, ())

        def acca_body(v, _):
            s = pl.ds(v * L, L)
            acca[s] = acca[s] + abuf[s]
            return ()
        lax.fori_loop(0, 4 * SL // L, acca_body, ())
        return ()
    lax.fori_loop(0, NW, part, ())

    def emit(v, _):
        d = jnp.maximum(accd[pl.ds(v * L, L)], 1.0)
        idx = (v * L + iota) * 4
        for k in range(4):
            val = acca[pl.ds(k * SL + v * L, L)] / d
            plsc.store_scatter(obuf, [idx + k], val)
        return ()
    lax.fori_loop(0, SL // L, emit, ())
    pltpu.sync_copy(obuf, mean_out.at[pl.ds(w * SL * 4, SL * 4)])


@functools.partial(
    pl.kernel,
    out_type=(
        jax.ShapeDtypeStruct((E_PAD,), _f32),
        jax.ShapeDtypeStruct((NC * N_PAD,), _f32),
    ),
    mesh=_MESH,
    compiler_params=_SC_PARAMS,
    scratch_types=[
        pltpu.VMEM((CH,), _i32),
        pltpu.VMEM((CH,), _i32),
        pltpu.VMEM((CH, C), _f32),
        pltpu.VMEM((CH, C), _f32),
        pltpu.VMEM((CH * 4,), _f32),
        pltpu.VMEM((CH,), _f32),
        pltpu.VMEM((N_PAD,), _f32),
        pltpu.VMEM((SL * 2,), _f32),
        pltpu.VMEM((768,), _f32),
        pltpu.VMEM((C * 6 * L,), _f32),
        pltpu.VMEM_SHARED((NS, N_PAD), _f32),
        pltpu.SemaphoreType.DMA,
        pltpu.SemaphoreType.DMA,
    ],
)
def _k1(xl_hbm, xr_hbm, src_hbm, dst_hbm, ea_hbm, wvec_hbm,
        logits_out, mmax_out,
        sidx, didx, rs, rt, eab, lgt, ml, rbuf, wstage, stab, shm, sem1, sem2):
    w = _wid()
    cid = lax.axis_index("c")
    sid = lax.axis_index("s")
    iota = _iota16()
    pltpu.sync_copy(wvec_hbm, wstage)

    def mksplat(c, _):
        for q, off in enumerate((0, C, 2 * C, 3 * C, 4 * C, 5 * C)):
            sp = plsc.load_gather(wstage, [jnp.full((L,), off, _i32) + c])
            stab[pl.ds((c * 6 + q) * L, L)] = sp
        return ()
    lax.fori_loop(0, C, mksplat, ())
    _fill(ml, N_PAD // L, -1e30)

    def chunk(ch, _):
        base = pl.multiple_of(w * EPT + ch * CH, CH)
        pltpu.sync_copy(src_hbm.at[pl.ds(base, CH)], sidx)
        pltpu.sync_copy(dst_hbm.at[pl.ds(base, CH)], didx)
        pltpu.sync_copy(ea_hbm.at[pl.ds(base * 4, CH * 4)], eab)
        cp1 = pltpu.async_copy(xl_hbm.at[sidx], rs, sem1)
        cp2 = pltpu.async_copy(xr_hbm.at[didx], rt, sem2)
        cp1.wait()
        cp2.wait()

        NG = 4

        def gpass(p, _):
            erow = [p * (NG * L) + g * L + iota for g in range(NG)]
            ea_g = []
            for g in range(NG):
                e4 = erow[g] * 4
                ea_g.append([plsc.load_gather(eab, [e4 + k])
                             for k in range(4)])
            acc6 = [jnp.zeros((L,), _f32)] * NG
            acc4 = [jnp.zeros((L,), _f32)] * NG

            def cblk(c0, carry):
                a6, a4 = carry
                a6 = list(a6)
                a4 = list(a4)
                U = 16
                for u in range(U):
                    cc = jnp.full((L,), c0 * U, _i32) + u
                    cb = c0 * (U * 6 * L) + u * 6 * L
                    w0 = stab[pl.ds(cb, L)]
                    w1 = stab[pl.ds(cb + L, L)]
                    w2 = stab[pl.ds(cb + 2 * L, L)]
                    w3 = stab[pl.ds(cb + 3 * L, L)]
                    s6 = stab[pl.ds(cb + 4 * L, L)]
                    s4 = stab[pl.ds(cb + 5 * L, L)]
                    for g in range(NG):
                        s = plsc.load_gather(rs, [erow[g], cc])
                        t = plsc.load_gather(rt, [erow[g], cc])
                        e = ea_g[g]
                        f = (e[0] * w0 + e[1] * w1) + (e[2] * w2 + e[3] * w3)
                        z = (s + t) + f
                        a6[g] = a6[g] + z * s6
                        a4[g] = a4[g] + jnp.abs(z) * s4
                return tuple(a6), tuple(a4)
            acc6, acc4 = lax.fori_loop(0, C // 16, cblk,
                                       (tuple(acc6), tuple(acc4)))
            for g in range(NG):
                logit = acc6[g] + acc4[g]
                lgt[pl.ds(p * (NG * L) + g * L, L)] = logit
                d16 = didx[pl.ds(p * (NG * L) + g * L, L)]
                cur = plsc.load_gather(ml, [d16])
                plsc.store_scatter(ml, [d16], jnp.maximum(cur, logit))
            return ()
        lax.fori_loop(0, CH // (NG * L), gpass, ())
        pltpu.sync_copy(lgt, logits_out.at[pl.ds(base, CH)])
        return ()
    lax.fori_loop(0, NCH, chunk, ())

    pltpu.sync_copy(ml, shm.at[sid])
    plsc.subcore_barrier()
    nbase = sid * (N_PAD // NS)

    def red(j, _):
        pltpu.sync_copy(shm.at[j, pl.ds(nbase, N_PAD // NS)],
                        rbuf.at[pl.ds(0, N_PAD // NS)])

        @pl.when(j == 0)
        def _():
            def cpy(v, _):
                s = pl.ds(v * L, L)
                ml[s] = rbuf[s]
                return ()
            lax.fori_loop(0, N_PAD // NS // L, cpy, ())

        @pl.when(j > 0)
        def _():
            def mx(v, _):
                s = pl.ds(v * L, L)
                ml[s] = jnp.maximum(ml[s], rbuf[s])
                return ()
            lax.fori_loop(0, N_PAD // NS // L, mx, ())
        return ()
    lax.fori_loop(0, NS, red, ())
    pltpu.sync_copy(ml.at[pl.ds(0, N_PAD // NS)],
                    mmax_out.at[pl.ds(cid * N_PAD + nbase, N_PAD // NS)])


@functools.partial(
    pl.kernel,
    out_type=(
        jax.ShapeDtypeStruct((E_PAD,), _f32),
        jax.ShapeDtypeStruct((NC * N_PAD,), _f32),
    ),
    mesh=_MESH,
    compiler_params=_SC_PARAMS,
    scratch_types=[
        pltpu.VMEM((N_PAD,), _f32),
        pltpu.VMEM((N_PAD,), _f32),
        pltpu.VMEM((N_PAD,), _f32),
        pltpu.VMEM((CH,), _i32),
        pltpu.VMEM((CH,), _f32),
        pltpu.VMEM((CH,), _f32),
        pltpu.VMEM_SHARED((NS, N_PAD), _f32),
    ],
)
def _k3(logits_hbm, dst_hbm, mmax_hbm, ex_out, dpart_out,
        ml, dl, rbuf, didx, lbuf, exbuf, shm):
    w = _wid()
    cid = lax.axis_index("c")
    sid = lax.axis_index("s")
    pltpu.sync_copy(mmax_hbm.at[pl.ds(0, N_PAD)], ml)
    pltpu.sync_copy(mmax_hbm.at[pl.ds(N_PAD, N_PAD)], rbuf)

    def mx(v, _):
        s = pl.ds(v * L, L)
        ml[s] = jnp.maximum(ml[s], rbuf[s])
        return ()
    lax.fori_loop(0, N_PAD // L, mx, ())
    _fill(dl, N_PAD // L, 0.0)

    def chunk(ch, _):
        base = pl.multiple_of(w * EPT + ch * CH, CH)
        pltpu.sync_copy(dst_hbm.at[pl.ds(base, CH)], didx)
        pltpu.sync_copy(logits_hbm.at[pl.ds(base, CH)], lbuf)

        def group(g, _):
            s = pl.ds(g * L, L)
            d16 = didx[s]
            mg = plsc.load_gather(ml, [d16])
            ex16 = jnp.exp(lbuf[s] - mg)
            exbuf[s] = ex16
            plsc.addupdate_scatter(dl, [d16], ex16)
            return ()
        lax.fori_loop(0, CH // L, group, ())
        pltpu.sync_copy(exbuf, ex_out.at[pl.ds(base, CH)])
        return ()
    lax.fori_loop(0, NCH, chunk, ())

    pltpu.sync_copy(dl, shm.at[sid])
    plsc.subcore_barrier()
    nbase = sid * (N_PAD // NS)

    def red(j, _):
        pltpu.sync_copy(shm.at[j, pl.ds(nbase, N_PAD // NS)],
                        rbuf.at[pl.ds(0, N_PAD // NS)])

        @pl.when(j == 0)
        def _():
            def cpy(v, _):
                s = pl.ds(v * L, L)
                dl[s] = rbuf[s]
                return ()
            lax.fori_loop(0, N_PAD // NS // L, cpy, ())

        @pl.when(j > 0)
        def _():
            def ad(v, _):
                s = pl.ds(v * L, L)
                dl[s] = dl[s] + rbuf[s]
                return ()
            lax.fori_loop(0, N_PAD // NS // L, ad, ())
        return ()
    lax.fori_loop(0, NS, red, ())
    pltpu.sync_copy(dl.at[pl.ds(0, N_PAD // NS)],
                    dpart_out.at[pl.ds(cid * N_PAD + nbase, N_PAD // NS)])


@functools.partial(
    pl.kernel,
    out_type=jax.ShapeDtypeStruct((NC * N_PAD, C), _f32),
    mesh=_MESH,
    compiler_params=_SC_PARAMS,
    scratch_types=[
        pltpu.VMEM((N_PAD,), _f32),
        pltpu.VMEM((N_PAD,), _f32),
        pltpu.VMEM((CH,), _i32),
        pltpu.VMEM((CH,), _i32),
        pltpu.VMEM((CH,), _f32),
        pltpu.VMEM((CH,), _f32),
        pltpu.VMEM((CH, C), _f32),
        pltpu.VMEM((64, C), _f32),
        pltpu.VMEM_SHARED((N_PAD, C), _f32),
        pltpu.SemaphoreType.DMA,
    ],
)
def _k5(xl_hbm, src_hbm, dst_hbm, ex_hbm, dpart_hbm, opart_out,
        divl, rbuf, sidx, didx, exb, alb, rs, zbuf, shacc, sem):
    w = _wid()
    cid = lax.axis_index("c")
    sid = lax.axis_index("s")
    pltpu.sync_copy(dpart_hbm.at[pl.ds(0, N_PAD)], divl)
    pltpu.sync_copy(dpart_hbm.at[pl.ds(N_PAD, N_PAD)], rbuf)

    def inv(v, _):
        s = pl.ds(v * L, L)
        divl[s] = 1.0 / ((divl[s] + rbuf[s]) + 1e-16)
        return ()
    lax.fori_loop(0, N_PAD // L, inv, ())

    def zfill(r, _):
        for cc in range(C // L):
            zbuf[r, pl.ds(cc * L, L)] = jnp.zeros((L,), _f32)
        return ()
    lax.fori_loop(0, 64, zfill, ())
    for t in range(N_PAD // NS // 64):
        pltpu.sync_copy(zbuf, shacc.at[pl.ds(sid * (N_PAD // NS) + t * 64, 64)])
    plsc.subcore_barrier()

    def chunk(ch, _):
        base = pl.multiple_of(w * EPT + ch * CH, CH)
        pltpu.sync_copy(src_hbm.at[pl.ds(base, CH)], sidx)
        pltpu.sync_copy(dst_hbm.at[pl.ds(base, CH)], didx)
        pltpu.sync_copy(ex_hbm.at[pl.ds(base, CH)], exb)
        pltpu.async_copy(xl_hbm.at[sidx], rs, sem).wait()

        def group(g, _):
            s = pl.ds(g * L, L)
            d16 = didx[s]
            dv = plsc.load_gather(divl, [d16])
            alb[s] = exb[s] * dv
            return ()
        lax.fori_loop(0, CH // L, group, ())

        def scale(e, _):
            a = plsc.load_gather(alb, [jnp.full((L,), e, _i32)])
            for j in range(C // L):
                s = pl.ds(j * L, L)
                rs[e, s] = rs[e, s] * a
            return ()
        lax.fori_loop(0, CH, scale, ())
        pltpu.sync_copy(rs, shacc.at[didx], add=True)
        return ()
    lax.fori_loop(0, NCH, chunk, ())

    plsc.subcore_barrier()
    nbase = sid * (N_PAD // NS)
    pltpu.sync_copy(shacc.at[pl.ds(nbase, N_PAD // NS)],
                    opart_out.at[pl.ds(cid * N_PAD + nbase, N_PAD // NS)])


ROW_BLK = 512


def _proj_first_body(x_ref, w_ref, b_ref, xl_ref, xr_ref):
    o = jnp.dot(x_ref[...], w_ref[...], preferred_element_type=_f32) + b_ref[...]
    xl_ref[...] = o[:, :C]
    xr_ref[...] = o[:, C:]


def _proj_next_body(p0_ref, p1_ref, cb_ref, w_ref, b_ref, xl_ref, xr_ref):
    h = jax.nn.relu(p0_ref[...] + p1_ref[...] + cb_ref[...])
    o = jnp.dot(h, w_ref[...], preferred_element_type=_f32) + b_ref[...]
    xl_ref[...] = o[:, :C]
    xr_ref[...] = o[:, C:]


def _proj_first(x_pad, W, b):
    grid = (N_PAD // ROW_BLK,)
    return pl.pallas_call(
        _proj_first_body,
        grid=grid,
        in_specs=[
            pl.BlockSpec((ROW_BLK, C), lambda i: (i, 0)),
            pl.BlockSpec((C, 2 * C), lambda i: (0, 0)),
            pl.BlockSpec((1, 2 * C), lambda i: (0, 0)),
        ],
        out_specs=[
            pl.BlockSpec((ROW_BLK, C), lambda i: (i, 0)),
            pl.BlockSpec((ROW_BLK, C), lambda i: (i, 0)),
        ],
        out_shape=[
            jax.ShapeDtypeStruct((N_PAD, C), _f32),
            jax.ShapeDtypeStruct((N_PAD, C), _f32),
        ],
    )(x_pad, W, b.reshape(1, -1))


def _proj_next(opart, cprev, W, b):
    grid = (N_PAD // ROW_BLK,)
    return pl.pallas_call(
        _proj_next_body,
        grid=grid,
        in_specs=[
            pl.BlockSpec((ROW_BLK, C), lambda i: (i, 0)),
            pl.BlockSpec((ROW_BLK, C), lambda i: (i + N_PAD // ROW_BLK, 0)),
            pl.BlockSpec((1, C), lambda i: (0, 0)),
            pl.BlockSpec((C, 2 * C), lambda i: (0, 0)),
            pl.BlockSpec((1, 2 * C), lambda i: (0, 0)),
        ],
        out_specs=[
            pl.BlockSpec((ROW_BLK, C), lambda i: (i, 0)),
            pl.BlockSpec((ROW_BLK, C), lambda i: (i, 0)),
        ],
        out_shape=[
            jax.ShapeDtypeStruct((N_PAD, C), _f32),
            jax.ShapeDtypeStruct((N_PAD, C), _f32),
        ],
    )(opart, opart, cprev.reshape(1, C), W, b.reshape(1, -1))


def _pool_body(p0_ref, p1_ref, cb_ref, bt_ref, wt_ref, btb_ref, o_ref,
               acc_ref, cnt_ref):
    i = pl.program_id(0)
    h = p0_ref[...] + p1_ref[...] + cb_ref[...]
    bb = bt_ref[...]
    iotag = lax.broadcasted_iota(_i32, (ROW_BLK, G), 1).astype(_f32)
    oh = (bb == iotag).astype(_f32)
    contrib = lax.dot_general(oh, h, (((0,), (0,)), ((), ())),
                              preferred_element_type=_f32)
    cntc = lax.dot_general(oh, jnp.ones_like(h), (((0,), (0,)), ((), ())),
                           preferred_element_type=_f32)

    @pl.when(i == 0)
    def _():
        acc_ref[...] = contrib
        cnt_ref[...] = cntc

    @pl.when(i > 0)
    def _():
        acc_ref[...] = acc_ref[...] + contrib
        cnt_ref[...] = cnt_ref[...] + cntc

    @pl.when(i == N_PAD // ROW_BLK - 1)
    def _():
        g = acc_ref[...] / jnp.maximum(cnt_ref[...], 1.0)
        o_ref[...] = (jnp.dot(g, wt_ref[...], preferred_element_type=_f32)
                      + btb_ref[...])


def _pool(opart3, c3, batch_f, Wt_pad, bt_pad):
    grid = (N_PAD // ROW_BLK,)
    return pl.pallas_call(
        _pool_body,
        grid=grid,
        in_specs=[
            pl.BlockSpec((ROW_BLK, C), lambda i: (i, 0)),
            pl.BlockSpec((ROW_BLK, C), lambda i: (i + N_PAD // ROW_BLK, 0)),
            pl.BlockSpec((1, C), lambda i: (0, 0)),
            pl.BlockSpec((ROW_BLK, 1), lambda i: (i, 0)),
            pl.BlockSpec((C, C), lambda i: (0, 0)),
            pl.BlockSpec((1, C), lambda i: (0, 0)),
        ],
        out_specs=pl.BlockSpec((G, C), lambda i: (0, 0)),
        out_shape=jax.ShapeDtypeStruct((G, C), _f32),
        scratch_shapes=[
            pltpu.VMEM((G, C), _f32),
            pltpu.VMEM((G, C), _f32),
        ],
    )(opart3, opart3, c3.reshape(1, C), batch_f, Wt_pad, bt_pad.reshape(1, C))


def _gatv2_layer(xl, xr, src, dst, ea_flat, We, att, bias):
    a = att.reshape(-1)
    wvec = jnp.concatenate([We.reshape(-1), 0.6 * a, 0.4 * a])
    logits, mmax = _k1(xl, xr, src, dst, ea_flat, wvec)
    ex, dpart = _k3(logits, dst, mmax)
    opart = _k5(xl, src, dst, ex, dpart)
    del bias
    return opart


def kernel(x, edge_index, edge_attr, batch, r_target, W1l, b1l, W1r, b1r, W1e, a1, c1, W2l, b2l, W2r, b2r, W2e, a2, c2, W3l, b3l, W3r, b3r, W3e, a3, c3, Wt, bt):
    del r_target
    src0 = edge_index[0].astype(_i32)
    dst0 = edge_index[1].astype(_i32)
    ea = edge_attr.astype(_f32)

    dst0_pad = jnp.concatenate([dst0, jnp.full((E0_PAD - E,), PADN, _i32)])
    ea0_flat = jnp.concatenate([ea.reshape(-1),
                                jnp.zeros(((E0_PAD - E) * 4,), _f32)])
    deg_part, asum_part = _k0(dst0_pad, ea0_flat)
    mean_flat = _k0b(deg_part, asum_part)

    loop = jnp.arange(N, dtype=_i32)
    src = jnp.concatenate([src0, loop, jnp.full((E_PAD - E_AUG,), PADN, _i32)])
    dst = jnp.concatenate([dst0, loop, jnp.full((E_PAD - E_AUG,), PADN, _i32)])
    ea_flat = jnp.concatenate([ea.reshape(-1), mean_flat[:N * 4],
                               jnp.zeros(((E_PAD - E_AUG) * 4,), _f32)])

    x_pad = jnp.pad(x, ((0, N_PAD - N), (0, 0)))

    xl, xr = _proj_first(x_pad, jnp.concatenate([W1l, W1r], axis=1),
                         jnp.concatenate([b1l, b1r]))
    op1 = _gatv2_layer(xl, xr, src, dst, ea_flat, W1e, a1, c1)

    xl, xr = _proj_next(op1, c1, jnp.concatenate([W2l, W2r], axis=1),
                        jnp.concatenate([b2l, b2r]))
    op2 = _gatv2_layer(xl, xr, src, dst, ea_flat, W2e, a2, c2)

    xl, xr = _proj_next(op2, c2, jnp.concatenate([W3l, W3r], axis=1),
                        jnp.concatenate([b3l, b3r]))
    op3 = _gatv2_layer(xl, xr, src, dst, ea_flat, W3e, a3, c3)

    batch_f = jnp.concatenate([batch.astype(_f32),
                               jnp.full((N_PAD - N,), float(G), _f32)])
    Wt_pad = jnp.pad(Wt, ((0, 0), (0, C - NT)))
    bt_pad = jnp.pad(bt, (0, C - NT))
    pooled = _pool(op3, c3, batch_f.reshape(N_PAD, 1), Wt_pad, bt_pad)
    return pooled[:, :NT]
